# baseline (speedup 1.0000x reference)
import functools

import jax
import jax.numpy as jnp
from jax import lax
from jax.experimental import pallas as pl
from jax.experimental.pallas import tpu as pltpu
from jax.experimental.pallas import tpu_sc as plsc

_K = 1024
_D = 256
_BN = 1024
_NW = 32
_PC = 32
_S = 4


def _tc_slice_kernel(x_ref, e_ref, xq_in_ref, xq_ref, idx_ref):
    del xq_in_ref
    x = x_ref[...]
    e = e_ref[...]
    u_sq = jnp.sum(jnp.square(x), axis=-1, keepdims=True)
    v_sq = jnp.sum(jnp.square(e), axis=-1)[None, :]
    dot = jax.lax.dot_general(
        x, e, (((1,), (1,)), ((), ())),
        preferred_element_type=jnp.float32)
    dist = u_sq + v_sq - 2.0 * dot
    m = jnp.min(dist, axis=-1, keepdims=True)
    iota_k = jax.lax.broadcasted_iota(jnp.int32, (x.shape[0], _K), 1)
    idx = jnp.min(jnp.where(dist == m, iota_k, _K), axis=-1, keepdims=True)
    idx_ref[...] = idx
    p = (iota_k == idx).astype(jnp.float32)
    xq_ref[...] = jax.lax.dot_general(
        p, e, (((1,), (0,)), ((), ())),
        preferred_element_type=jnp.float32)


def _tc_slice(x_nd, embeddings, xq_buf, block0, nsl):
    n = x_nd.shape[0]
    return pl.pallas_call(
        _tc_slice_kernel,
        grid=(nsl // _BN,),
        in_specs=[
            pl.BlockSpec((_BN, _D), lambda i: (block0 + i, 0)),
            pl.BlockSpec((_K, _D), lambda i: (0, 0)),
            pl.BlockSpec(memory_space=pl.ANY),
        ],
        out_specs=[
            pl.BlockSpec((_BN, _D), lambda i: (block0 + i, 0)),
            pl.BlockSpec((_BN, 1), lambda i: (i, 0)),
        ],
        out_shape=[
            jax.ShapeDtypeStruct((n, _D), jnp.float32),
            jax.ShapeDtypeStruct((nsl, 1), jnp.int32),
        ],
        input_output_aliases={2: 0},
    )(x_nd, embeddings, xq_buf)


def _sc_p_writer(idx_s, zeros_t, p_ref, row0, nsl):
    b_per_w = nsl // _NW
    n_pc = b_per_w // _PC
    mesh = plsc.VectorSubcoreMesh(core_axis_name="c", subcore_axis_name="s")

    @functools.partial(
        pl.kernel, mesh=mesh,
        out_type=(),
        compiler_params=pltpu.CompilerParams(needs_layout_passes=False),
        scratch_types=[
            pltpu.VMEM((b_per_w,), jnp.int32),
            pltpu.VMEM((_PC * _K,), jnp.float32),
            pltpu.VMEM((_PC * _K,), jnp.float32),
            pltpu.SemaphoreType.DMA,
            pltpu.SemaphoreType.DMA,
        ],
    )
    def p_writer(idx_hbm, zeros_hbm, p_hbm, idx_v, pbuf0, pbuf1, psem0, psem1):
        wid = lax.axis_index("s") * 2 + lax.axis_index("c")
        base_tok = wid * b_per_w
        pltpu.sync_copy(idx_hbm.at[pl.ds(base_tok, b_per_w)], idx_v)
        pltpu.sync_copy(zeros_hbm, pbuf0)
        pltpu.sync_copy(zeros_hbm, pbuf1)
        pbufs = (pbuf0, pbuf1)
        psems = (psem0, psem1)
        ones = jnp.full((16,), 1.0, jnp.float32)
        zeros16 = jnp.zeros((16,), jnp.float32)
        lane = lax.iota(jnp.int32, 16)
        prev = [None, None]
        for c in range(n_pc):
            buf = pbufs[c % 2]
            if prev[c % 2] is not None:
                cp_prev, pos_prev = prev[c % 2]
                cp_prev.wait()
                for pp in pos_prev:
                    plsc.store_scatter(buf, [pp], zeros16)
            positions = []
            for t in range(_PC // 16):
                tok = c * _PC + t * 16
                iv = plsc.load_gather(idx_v, [tok + lane])
                pos = (t * 16 + lane) * _K + iv
                plsc.store_scatter(buf, [pos], ones)
                positions.append(pos)
            out_off = (row0 + base_tok + c * _PC) * _K
            cp = pltpu.async_copy(
                buf, p_hbm.at[pl.ds(out_off, _PC * _K)], psems[c % 2])
            prev[c % 2] = (cp, positions)
        for pr in prev:
            if pr is not None:
                pr[0].wait()

    return p_writer(idx_s, zeros_t, p_ref)


def kernel(x__d, embeddings):
    input_size = x__d.shape[:-1]
    d = x__d.shape[-1]
    x_nd = x__d.reshape(-1, d)
    n = x_nd.shape[0]
    nsl = n // _S
    zeros_t = jnp.zeros((_PC * _K,), jnp.float32)
    p_ref = jax.new_ref(jnp.zeros((n * _K,), jnp.float32))
    xq_buf = jnp.zeros((n, _D), jnp.float32)
    for s in range(_S):
        xq_buf, idx_s = _tc_slice(x_nd, embeddings, xq_buf,
                                  s * (nsl // _BN), nsl)
        _sc_p_writer(idx_s.reshape(nsl), zeros_t, p_ref, s * nsl, nsl)
    p_flat = p_ref[...]
    xq__d = xq_buf.reshape(input_size + (d,))
    p__k = p_flat.reshape(input_size + (_K,))
    return (xq__d, p__k)


# speedup vs baseline: 3.5106x; 3.5106x over previous
import jax
import jax.numpy as jnp
from jax.experimental import pallas as pl

_K = 1024
_D = 256
_BN = 2048


def _vq_block_kernel(x_ref, e_ref, xq_ref, p_ref):
    x = x_ref[...]
    e = e_ref[...]
    u_sq = jnp.sum(jnp.square(x), axis=-1, keepdims=True)
    v_sq = jnp.sum(jnp.square(e), axis=-1)[None, :]
    dot = jax.lax.dot_general(
        x, e, (((1,), (1,)), ((), ())),
        preferred_element_type=jnp.float32)
    dist = u_sq + v_sq - 2.0 * dot
    m = jnp.min(dist, axis=-1, keepdims=True)
    iota_k = jax.lax.broadcasted_iota(jnp.int32, (x.shape[0], _K), 1)
    idx = jnp.min(jnp.where(dist == m, iota_k, _K), axis=-1, keepdims=True)
    p = (iota_k == idx).astype(jnp.float32)
    p_ref[...] = p
    xq_ref[...] = jax.lax.dot_general(
        p, e, (((1,), (0,)), ((), ())),
        preferred_element_type=jnp.float32)


def kernel(x__d, embeddings):
    input_size = x__d.shape[:-1]
    d = x__d.shape[-1]
    x_nd = x__d.reshape(-1, d)
    n = x_nd.shape[0]
    grid = (n // _BN,)
    xq_nd, p_nk = pl.pallas_call(
        _vq_block_kernel,
        grid=grid,
        in_specs=[
            pl.BlockSpec((_BN, _D), lambda i: (i, 0)),
            pl.BlockSpec((_K, _D), lambda i: (0, 0)),
        ],
        out_specs=[
            pl.BlockSpec((_BN, _D), lambda i: (i, 0)),
            pl.BlockSpec((_BN, _K), lambda i: (i, 0)),
        ],
        out_shape=[
            jax.ShapeDtypeStruct((n, _D), jnp.float32),
            jax.ShapeDtypeStruct((n, _K), jnp.float32),
        ],
    )(x_nd, embeddings)
    xq__d = xq_nd.reshape(input_size + (d,))
    p__k = p_nk.reshape(input_size + (_K,))
    return (xq__d, p__k)


# speedup vs baseline: 3.5713x; 1.0173x over previous
import jax
import jax.numpy as jnp
from jax.experimental import pallas as pl

_K = 1024
_D = 256
_BN = 3072


def _vq_block_kernel(x_ref, e_ref, xq_ref, p_ref):
    x = x_ref[...]
    e = e_ref[...]
    u_sq = jnp.sum(jnp.square(x), axis=-1, keepdims=True)
    v_sq = jnp.sum(jnp.square(e), axis=-1)[None, :]
    dot = jax.lax.dot_general(
        x, e, (((1,), (1,)), ((), ())),
        preferred_element_type=jnp.float32)
    dist = u_sq + v_sq - 2.0 * dot
    m = jnp.min(dist, axis=-1, keepdims=True)
    iota_k = jax.lax.broadcasted_iota(jnp.int32, (x.shape[0], _K), 1)
    idx = jnp.min(jnp.where(dist == m, iota_k, _K), axis=-1, keepdims=True)
    p = (iota_k == idx).astype(jnp.float32)
    p_ref[...] = p
    xq_ref[...] = jax.lax.dot_general(
        p, e, (((1,), (0,)), ((), ())),
        preferred_element_type=jnp.float32)


def kernel(x__d, embeddings):
    input_size = x__d.shape[:-1]
    d = x__d.shape[-1]
    x_nd = x__d.reshape(-1, d)
    n = x_nd.shape[0]
    grid = (n // _BN,)
    xq_nd, p_nk = pl.pallas_call(
        _vq_block_kernel,
        grid=grid,
        in_specs=[
            pl.BlockSpec((_BN, _D), lambda i: (i, 0)),
            pl.BlockSpec((_K, _D), lambda i: (0, 0)),
        ],
        out_specs=[
            pl.BlockSpec((_BN, _D), lambda i: (i, 0)),
            pl.BlockSpec((_BN, _K), lambda i: (i, 0)),
        ],
        out_shape=[
            jax.ShapeDtypeStruct((n, _D), jnp.float32),
            jax.ShapeDtypeStruct((n, _K), jnp.float32),
        ],
    )(x_nd, embeddings)
    xq__d = xq_nd.reshape(input_size + (d,))
    p__k = p_nk.reshape(input_size + (_K,))
    return (xq__d, p__k)
